# baseline (speedup 1.0000x reference)
import jax
import jax.numpy as jnp
from jax.experimental import pallas as pl
from jax.experimental.pallas import tpu as pltpu

_SCALE = 80.0
_R2 = 40.0 * 40.0
_BIG = 1e10
_TN = 512


def _split3_bf16(x):
    s1 = x.astype(jnp.bfloat16)
    r1 = x - s1.astype(jnp.float32)
    s2 = r1.astype(jnp.bfloat16)
    r2 = r1 - s2.astype(jnp.float32)
    s3 = r2.astype(jnp.bfloat16)
    return s1, s2, s3


def _chamfer_kernel(p_ref, gt_ref, a1_ref, a2_ref, a2p_ref,
                    ge_ref, gep_ref, g2_ref, out_ref):
    N = p_ref.shape[1]
    M = gt_ref.shape[2]

    gx = gt_ref[0, 0:1, :] * _SCALE
    gy = gt_ref[0, 1:2, :] * _SCALE
    gz = gt_ref[0, 2:3, :] * _SCALE
    bb = gx * gx + gy * gy + gz * gz
    mg = bb < _R2
    ge = ge_ref[0]
    gep = gep_ref[0]
    g2 = g2_ref[0]

    def dotf(a, b):
        return jax.lax.dot_general(a, b, (((1,), (0,)), ((), ())),
                                   preferred_element_type=jnp.float32)

    def body(j, carry):
        cmin_u, cmin_m, rsum_u, rsum_m, cnt_p = carry
        p_blk = p_ref[0, pl.ds(j * _TN, _TN), :] * _SCALE
        px = p_blk[:, 0:1]
        py = p_blk[:, 1:2]
        pz = p_blk[:, 2:3]
        aa = px * px + py * py + pz * pz
        mp = aa < _R2

        a1 = a1_ref[0, pl.ds(j * _TN, _TN), :]
        a2 = a2_ref[0, pl.ds(j * _TN, _TN), :]
        a2p = a2p_ref[0, pl.ds(j * _TN, _TN), :]

        e = dotf(a1, ge)
        ep = dotf(a1, gep)
        f = dotf(a2, g2)
        fp = dotf(a2p, g2)

        rmin_u = jnp.maximum(aa + jnp.min(e, axis=1, keepdims=True), 0.0)
        rmin_m = jnp.maximum(aa + jnp.min(ep, axis=1, keepdims=True), 0.0)
        cmin_u = jnp.minimum(cmin_u, jnp.min(f, axis=0, keepdims=True))
        cmin_m = jnp.minimum(cmin_m, jnp.min(fp, axis=0, keepdims=True))

        rsum_u = rsum_u + jnp.sum(rmin_u)
        rsum_m = rsum_m + jnp.sum(jnp.where(mp, rmin_m, 0.0))
        cnt_p = cnt_p + jnp.sum(mp.astype(jnp.float32))
        return cmin_u, cmin_m, rsum_u, rsum_m, cnt_p

    init = (
        jnp.full((1, M), _BIG, jnp.float32),
        jnp.full((1, M), _BIG, jnp.float32),
        jnp.float32(0.0),
        jnp.float32(0.0),
        jnp.float32(0.0),
    )
    cmin_u, cmin_m, rsum_u, rsum_m, cnt_p = jax.lax.fori_loop(
        0, N // _TN, body, init)

    cmin_u = jnp.maximum(bb + cmin_u, 0.0)
    cmin_m = jnp.maximum(bb + cmin_m, 0.0)
    sum_c_u = jnp.sum(cmin_u)
    sum_c_m = jnp.sum(jnp.where(mg, cmin_m, 0.0))
    cnt_g = jnp.sum(mg.astype(jnp.float32))

    non_filtered = rsum_u / N + sum_c_u / M
    filtered = (rsum_m / jnp.maximum(cnt_p, 1.0)
                + sum_c_m / jnp.maximum(cnt_g, 1.0))
    loss = 0.7 * filtered + 0.3 * non_filtered
    out_ref[:, :, :] = jnp.broadcast_to(loss, (1, 1, 1))


def kernel(image_pred, image_gt):
    B, N, _ = image_pred.shape
    M = image_gt.shape[1]
    gt_t = jnp.swapaxes(image_gt, 1, 2)

    p = image_pred * _SCALE
    g = image_gt * _SCALE
    p_r = p.astype(jnp.bfloat16)
    g_r = g.astype(jnp.bfloat16)

    aa = jnp.sum(p * p, axis=-1, keepdims=True)
    bb = jnp.sum(g * g, axis=-1, keepdims=True)
    pen_p = jnp.where(aa < _R2, 0.0, _BIG)
    pen_g = jnp.where(bb < _R2, 0.0, _BIG)

    ones_n = jnp.ones((B, N, 1), jnp.bfloat16)
    zeros_n = jnp.zeros((B, N, 2), jnp.bfloat16)

    def stack3(x):
        s1, s2, s3 = _split3_bf16(x)
        return [s1, s2, s3]

    a1 = jnp.concatenate([p_r, ones_n, ones_n, ones_n, zeros_n], axis=-1)
    a2 = jnp.concatenate([p_r] + stack3(aa) + [zeros_n], axis=-1)
    a2p = jnp.concatenate([p_r] + stack3(aa + pen_p) + [zeros_n], axis=-1)

    g2t = jnp.swapaxes(-2.0 * g_r.astype(jnp.float32), 1, 2
                       ).astype(jnp.bfloat16)
    ones_m = jnp.ones((B, 1, M), jnp.bfloat16)
    zeros_m = jnp.zeros((B, 2, M), jnp.bfloat16)

    def stack3_t(x):
        return [jnp.swapaxes(s, 1, 2) for s in _split3_bf16(x)]

    ge = jnp.concatenate([g2t] + stack3_t(bb) + [zeros_m], axis=1)
    gep = jnp.concatenate([g2t] + stack3_t(bb + pen_g) + [zeros_m], axis=1)
    g2 = jnp.concatenate([g2t, ones_m, ones_m, ones_m, zeros_m], axis=1)

    per_batch = pl.pallas_call(
        _chamfer_kernel,
        grid=(B,),
        in_specs=[
            pl.BlockSpec((1, N, 3), lambda b: (b, 0, 0)),
            pl.BlockSpec((1, 3, M), lambda b: (b, 0, 0)),
            pl.BlockSpec((1, N, 8), lambda b: (b, 0, 0)),
            pl.BlockSpec((1, N, 8), lambda b: (b, 0, 0)),
            pl.BlockSpec((1, N, 8), lambda b: (b, 0, 0)),
            pl.BlockSpec((1, 8, M), lambda b: (b, 0, 0)),
            pl.BlockSpec((1, 8, M), lambda b: (b, 0, 0)),
            pl.BlockSpec((1, 8, M), lambda b: (b, 0, 0)),
        ],
        out_specs=pl.BlockSpec((1, 1, 1), lambda b: (b, 0, 0)),
        out_shape=jax.ShapeDtypeStruct((B, 1, 1), jnp.float32),
        compiler_params=pltpu.CompilerParams(
            dimension_semantics=("parallel",)),
    )(image_pred, gt_t, a1, a2, a2p, ge, gep, g2)
    return jnp.mean(per_batch)


# speedup vs baseline: 1.5176x; 1.5176x over previous
import jax
import jax.numpy as jnp
from jax.experimental import pallas as pl
from jax.experimental.pallas import tpu as pltpu

_SCALE = 80.0
_R2 = 40.0 * 40.0
_BIG = 1e10
_TN = 512


def _chamfer_kernel(p_ref, gt_ref, pr_ref, g2_ref, out_ref):
    N = p_ref.shape[1]
    M = gt_ref.shape[2]

    gx = gt_ref[0, 0:1, :] * _SCALE
    gy = gt_ref[0, 1:2, :] * _SCALE
    gz = gt_ref[0, 2:3, :] * _SCALE
    bb = gx * gx + gy * gy + gz * gz
    mg = bb < _R2
    bbm = jnp.where(mg, bb, bb + _BIG)
    g2 = g2_ref[0]

    def body(j, carry):
        cmin_u, cmin_m, rsum_u, rsum_m, cnt_p = carry
        p_blk = p_ref[0, pl.ds(j * _TN, _TN), :] * _SCALE
        px = p_blk[:, 0:1]
        py = p_blk[:, 1:2]
        pz = p_blk[:, 2:3]
        aa = px * px + py * py + pz * pz
        mp = aa < _R2
        aam = jnp.where(mp, aa, aa + _BIG)

        p_r = pr_ref[0, pl.ds(j * _TN, _TN), :]
        ab2 = jax.lax.dot_general(
            p_r, g2, (((1,), (0,)), ((), ())),
            preferred_element_type=jnp.float32)

        rmin_u = jnp.maximum(
            aa + jnp.min(bb + ab2, axis=1, keepdims=True), 0.0)
        rmin_m = jnp.maximum(
            aa + jnp.min(bbm + ab2, axis=1, keepdims=True), 0.0)
        cmin_u = jnp.minimum(cmin_u, jnp.min(aa + ab2, axis=0, keepdims=True))
        cmin_m = jnp.minimum(cmin_m, jnp.min(aam + ab2, axis=0, keepdims=True))

        rsum_u = rsum_u + jnp.sum(rmin_u)
        rsum_m = rsum_m + jnp.sum(jnp.where(mp, rmin_m, 0.0))
        cnt_p = cnt_p + jnp.sum(mp.astype(jnp.float32))
        return cmin_u, cmin_m, rsum_u, rsum_m, cnt_p

    init = (
        jnp.full((1, M), _BIG, jnp.float32),
        jnp.full((1, M), _BIG, jnp.float32),
        jnp.float32(0.0),
        jnp.float32(0.0),
        jnp.float32(0.0),
    )
    cmin_u, cmin_m, rsum_u, rsum_m, cnt_p = jax.lax.fori_loop(
        0, N // _TN, body, init)

    cmin_u = jnp.maximum(bb + cmin_u, 0.0)
    cmin_m = jnp.maximum(bb + cmin_m, 0.0)
    sum_c_u = jnp.sum(cmin_u)
    sum_c_m = jnp.sum(jnp.where(mg, cmin_m, 0.0))
    cnt_g = jnp.sum(mg.astype(jnp.float32))

    non_filtered = rsum_u / N + sum_c_u / M
    filtered = (rsum_m / jnp.maximum(cnt_p, 1.0)
                + sum_c_m / jnp.maximum(cnt_g, 1.0))
    loss = 0.7 * filtered + 0.3 * non_filtered
    out_ref[:, :, :] = jnp.broadcast_to(loss, (1, 1, 1))


def kernel(image_pred, image_gt):
    B, N, _ = image_pred.shape
    M = image_gt.shape[1]
    gt_t = jnp.swapaxes(image_gt, 1, 2)

    p_r = (image_pred * _SCALE).astype(jnp.bfloat16)
    g2 = (-2.0 * (image_gt * _SCALE).astype(jnp.bfloat16)
          .astype(jnp.float32)).astype(jnp.bfloat16)
    p_r = jnp.pad(p_r, ((0, 0), (0, 0), (0, 5)))
    g2t = jnp.pad(jnp.swapaxes(g2, 1, 2),
                  ((0, 0), (0, 5), (0, 0)))

    per_batch = pl.pallas_call(
        _chamfer_kernel,
        grid=(B,),
        in_specs=[
            pl.BlockSpec((1, N, 3), lambda b: (b, 0, 0)),
            pl.BlockSpec((1, 3, M), lambda b: (b, 0, 0)),
            pl.BlockSpec((1, N, 8), lambda b: (b, 0, 0)),
            pl.BlockSpec((1, 8, M), lambda b: (b, 0, 0)),
        ],
        out_specs=pl.BlockSpec((1, 1, 1), lambda b: (b, 0, 0)),
        out_shape=jax.ShapeDtypeStruct((B, 1, 1), jnp.float32),
        compiler_params=pltpu.CompilerParams(
            dimension_semantics=("parallel",)),
    )(image_pred, gt_t, p_r, g2t)
    return jnp.mean(per_batch)


# speedup vs baseline: 1.5768x; 1.0390x over previous
import jax
import jax.numpy as jnp
from jax.experimental import pallas as pl
from jax.experimental.pallas import tpu as pltpu

_SCALE = 80.0
_R2 = 40.0 * 40.0
_BIG = 1e10
_TN = 1024


def _chamfer_kernel(p_ref, gt_ref, pr_ref, g2_ref, out_ref):
    N = p_ref.shape[1]
    M = gt_ref.shape[2]

    gx = gt_ref[0, 0:1, :] * _SCALE
    gy = gt_ref[0, 1:2, :] * _SCALE
    gz = gt_ref[0, 2:3, :] * _SCALE
    bb = gx * gx + gy * gy + gz * gz
    mg = bb < _R2
    bbm = jnp.where(mg, bb, bb + _BIG)
    g2 = g2_ref[0]

    def body(j, carry):
        cmin_u, cmin_m, rsum_u, rsum_m, cnt_p = carry
        p_blk = p_ref[0, pl.ds(j * _TN, _TN), :] * _SCALE
        px = p_blk[:, 0:1]
        py = p_blk[:, 1:2]
        pz = p_blk[:, 2:3]
        aa = px * px + py * py + pz * pz
        mp = aa < _R2
        aam = jnp.where(mp, aa, aa + _BIG)

        p_r = pr_ref[0, pl.ds(j * _TN, _TN), :]
        ab2 = jax.lax.dot_general(
            p_r, g2, (((1,), (0,)), ((), ())),
            preferred_element_type=jnp.float32)

        rmin_u = jnp.maximum(
            aa + jnp.min(bb + ab2, axis=1, keepdims=True), 0.0)
        rmin_m = jnp.maximum(
            aa + jnp.min(bbm + ab2, axis=1, keepdims=True), 0.0)
        cmin_u = jnp.minimum(cmin_u, jnp.min(aa + ab2, axis=0, keepdims=True))
        cmin_m = jnp.minimum(cmin_m, jnp.min(aam + ab2, axis=0, keepdims=True))

        rsum_u = rsum_u + jnp.sum(rmin_u)
        rsum_m = rsum_m + jnp.sum(jnp.where(mp, rmin_m, 0.0))
        cnt_p = cnt_p + jnp.sum(mp.astype(jnp.float32))
        return cmin_u, cmin_m, rsum_u, rsum_m, cnt_p

    init = (
        jnp.full((1, M), _BIG, jnp.float32),
        jnp.full((1, M), _BIG, jnp.float32),
        jnp.float32(0.0),
        jnp.float32(0.0),
        jnp.float32(0.0),
    )
    cmin_u, cmin_m, rsum_u, rsum_m, cnt_p = jax.lax.fori_loop(
        0, N // _TN, body, init)

    cmin_u = jnp.maximum(bb + cmin_u, 0.0)
    cmin_m = jnp.maximum(bb + cmin_m, 0.0)
    sum_c_u = jnp.sum(cmin_u)
    sum_c_m = jnp.sum(jnp.where(mg, cmin_m, 0.0))
    cnt_g = jnp.sum(mg.astype(jnp.float32))

    non_filtered = rsum_u / N + sum_c_u / M
    filtered = (rsum_m / jnp.maximum(cnt_p, 1.0)
                + sum_c_m / jnp.maximum(cnt_g, 1.0))
    loss = 0.7 * filtered + 0.3 * non_filtered
    out_ref[:, :, :] = jnp.broadcast_to(loss, (1, 1, 1))


def kernel(image_pred, image_gt):
    B, N, _ = image_pred.shape
    M = image_gt.shape[1]
    gt_t = jnp.swapaxes(image_gt, 1, 2)

    p_r = (image_pred * _SCALE).astype(jnp.bfloat16)
    g2 = (-2.0 * (image_gt * _SCALE).astype(jnp.bfloat16)
          .astype(jnp.float32)).astype(jnp.bfloat16)
    p_r = jnp.pad(p_r, ((0, 0), (0, 0), (0, 5)))
    g2t = jnp.pad(jnp.swapaxes(g2, 1, 2),
                  ((0, 0), (0, 5), (0, 0)))

    per_batch = pl.pallas_call(
        _chamfer_kernel,
        grid=(B,),
        in_specs=[
            pl.BlockSpec((1, N, 3), lambda b: (b, 0, 0)),
            pl.BlockSpec((1, 3, M), lambda b: (b, 0, 0)),
            pl.BlockSpec((1, N, 8), lambda b: (b, 0, 0)),
            pl.BlockSpec((1, 8, M), lambda b: (b, 0, 0)),
        ],
        out_specs=pl.BlockSpec((1, 1, 1), lambda b: (b, 0, 0)),
        out_shape=jax.ShapeDtypeStruct((B, 1, 1), jnp.float32),
        compiler_params=pltpu.CompilerParams(
            dimension_semantics=("parallel",)),
    )(image_pred, gt_t, p_r, g2t)
    return jnp.mean(per_batch)


# speedup vs baseline: 1.6208x; 1.0279x over previous
import jax
import jax.numpy as jnp
from jax.experimental import pallas as pl
from jax.experimental.pallas import tpu as pltpu

_SCALE = 80.0
_R2 = 40.0 * 40.0
_BIG = 1e10
_TN = 2048


def _chamfer_kernel(p_ref, gt_ref, pr_ref, g2_ref, out_ref):
    N = p_ref.shape[1]
    M = gt_ref.shape[2]

    gx = gt_ref[0, 0:1, :] * _SCALE
    gy = gt_ref[0, 1:2, :] * _SCALE
    gz = gt_ref[0, 2:3, :] * _SCALE
    bb = gx * gx + gy * gy + gz * gz
    mg = bb < _R2
    bbm = jnp.where(mg, bb, bb + _BIG)
    g2 = g2_ref[0]

    def body(j, carry):
        cmin_u, cmin_m, rsum_u, rsum_m, cnt_p = carry
        p_blk = p_ref[0, pl.ds(j * _TN, _TN), :] * _SCALE
        px = p_blk[:, 0:1]
        py = p_blk[:, 1:2]
        pz = p_blk[:, 2:3]
        aa = px * px + py * py + pz * pz
        mp = aa < _R2
        aam = jnp.where(mp, aa, aa + _BIG)

        p_r = pr_ref[0, pl.ds(j * _TN, _TN), :]
        ab2 = jax.lax.dot_general(
            p_r, g2, (((1,), (0,)), ((), ())),
            preferred_element_type=jnp.float32)

        rmin_u = jnp.maximum(
            aa + jnp.min(bb + ab2, axis=1, keepdims=True), 0.0)
        rmin_m = jnp.maximum(
            aa + jnp.min(bbm + ab2, axis=1, keepdims=True), 0.0)
        cmin_u = jnp.minimum(cmin_u, jnp.min(aa + ab2, axis=0, keepdims=True))
        cmin_m = jnp.minimum(cmin_m, jnp.min(aam + ab2, axis=0, keepdims=True))

        rsum_u = rsum_u + jnp.sum(rmin_u)
        rsum_m = rsum_m + jnp.sum(jnp.where(mp, rmin_m, 0.0))
        cnt_p = cnt_p + jnp.sum(mp.astype(jnp.float32))
        return cmin_u, cmin_m, rsum_u, rsum_m, cnt_p

    init = (
        jnp.full((1, M), _BIG, jnp.float32),
        jnp.full((1, M), _BIG, jnp.float32),
        jnp.float32(0.0),
        jnp.float32(0.0),
        jnp.float32(0.0),
    )
    cmin_u, cmin_m, rsum_u, rsum_m, cnt_p = jax.lax.fori_loop(
        0, N // _TN, body, init)

    cmin_u = jnp.maximum(bb + cmin_u, 0.0)
    cmin_m = jnp.maximum(bb + cmin_m, 0.0)
    sum_c_u = jnp.sum(cmin_u)
    sum_c_m = jnp.sum(jnp.where(mg, cmin_m, 0.0))
    cnt_g = jnp.sum(mg.astype(jnp.float32))

    non_filtered = rsum_u / N + sum_c_u / M
    filtered = (rsum_m / jnp.maximum(cnt_p, 1.0)
                + sum_c_m / jnp.maximum(cnt_g, 1.0))
    loss = 0.7 * filtered + 0.3 * non_filtered
    out_ref[:, :, :] = jnp.broadcast_to(loss, (1, 1, 1))


def kernel(image_pred, image_gt):
    B, N, _ = image_pred.shape
    M = image_gt.shape[1]
    gt_t = jnp.swapaxes(image_gt, 1, 2)

    p_r = (image_pred * _SCALE).astype(jnp.bfloat16)
    g2 = (-2.0 * (image_gt * _SCALE).astype(jnp.bfloat16)
          .astype(jnp.float32)).astype(jnp.bfloat16)
    p_r = jnp.pad(p_r, ((0, 0), (0, 0), (0, 5)))
    g2t = jnp.pad(jnp.swapaxes(g2, 1, 2),
                  ((0, 0), (0, 5), (0, 0)))

    per_batch = pl.pallas_call(
        _chamfer_kernel,
        grid=(B,),
        in_specs=[
            pl.BlockSpec((1, N, 3), lambda b: (b, 0, 0)),
            pl.BlockSpec((1, 3, M), lambda b: (b, 0, 0)),
            pl.BlockSpec((1, N, 8), lambda b: (b, 0, 0)),
            pl.BlockSpec((1, 8, M), lambda b: (b, 0, 0)),
        ],
        out_specs=pl.BlockSpec((1, 1, 1), lambda b: (b, 0, 0)),
        out_shape=jax.ShapeDtypeStruct((B, 1, 1), jnp.float32),
        compiler_params=pltpu.CompilerParams(
            dimension_semantics=("parallel",)),
    )(image_pred, gt_t, p_r, g2t)
    return jnp.mean(per_batch)
